# Initial kernel scaffold; baseline (speedup 1.0000x reference)
import jax
import jax.numpy as jnp
from jax.experimental import pallas as pl


def kernel(x, edge_index, etypes, weight, w_comp, loop_weight, h_bias):
    raise NotImplementedError("write your pallas kernel here")


# speedup vs baseline: 3.4812x; 3.4812x over previous
import functools

import jax
import jax.numpy as jnp
from jax import lax
from jax.experimental import pallas as pl
from jax.experimental.pallas import tpu as pltpu
from jax.experimental.pallas import tpu_sc as plsc

N = 10000
E = 320000
D_IN = 128
D_OUT = 128
R = 16
B = 8

NC = 2
NS = 16
NW = NC * NS
CH = 128
NCHUNK = E // CH
MAXK = -(-NCHUNK // NW)
NPAD = 10240
RPT = NPAD // NS

NB = 400


def _transform_body(x_ref, w_ref, wc_ref, h_ref):
    w = w_ref[...].reshape(B, D_IN * D_OUT)
    wc = wc_ref[...]
    big_w = jnp.dot(wc, w, preferred_element_type=jnp.float32)
    big_w = big_w.reshape(R, D_IN, D_OUT)
    xb = x_ref[...]
    for r in range(R):
        h_ref[r] = jnp.dot(xb, big_w[r], preferred_element_type=jnp.float32)


def _combine_body(p_ref, x_ref, lw_ref, b_ref, o_ref):
    loop = jnp.dot(x_ref[...], lw_ref[...], preferred_element_type=jnp.float32)
    o_ref[...] = p_ref[0] + p_ref[1] + loop + b_ref[0]


def _sc_gather_scatter(hflat, gidx, dstidx, zinit, partial,
                       idx_v, dst_v, rows_v, acc, sem):
    c = lax.axis_index("c")
    s = lax.axis_index("s")
    w = s * NC + c
    row0 = pl.multiple_of(s * RPT, RPT)

    pltpu.sync_copy(zinit.at[pl.ds(row0, RPT)], acc.at[pl.ds(row0, RPT)])
    plsc.subcore_barrier()

    def body(k, carry):
        chunk = w + NW * k

        @pl.when(chunk < NCHUNK)
        def _():
            base = pl.multiple_of(chunk * CH, CH)
            pltpu.sync_copy(gidx.at[pl.ds(base, CH)], idx_v)
            pltpu.sync_copy(dstidx.at[pl.ds(base, CH)], dst_v)
            pltpu.async_copy(hflat.at[idx_v], rows_v, sem).wait()
            pltpu.sync_copy(rows_v, acc.at[dst_v], add=True)

        return carry

    lax.fori_loop(0, MAXK, body, 0)
    plsc.subcore_barrier()

    pltpu.sync_copy(acc.at[pl.ds(row0, RPT)], partial.at[c, pl.ds(row0, RPT)])


def kernel(x, edge_index, etypes, weight, w_comp, loop_weight, h_bias):
    src = edge_index[0]
    dst = edge_index[1]
    gidx = etypes * jnp.int32(N) + src

    h_all = pl.pallas_call(
        _transform_body,
        grid=(N // NB,),
        in_specs=[
            pl.BlockSpec((NB, D_IN), lambda i: (i, 0)),
            pl.BlockSpec((B, D_IN, D_OUT), lambda i: (0, 0, 0)),
            pl.BlockSpec((R, B), lambda i: (0, 0)),
        ],
        out_specs=pl.BlockSpec((R, NB, D_OUT), lambda i: (0, i, 0)),
        out_shape=jax.ShapeDtypeStruct((R, N, D_OUT), jnp.float32),
    )(x, weight, w_comp)
    hflat = h_all.reshape(R * N, D_OUT)

    zinit = jnp.zeros((NPAD, D_OUT), jnp.float32)

    mesh = plsc.VectorSubcoreMesh(
        core_axis_name="c", subcore_axis_name="s", num_cores=NC, num_subcores=NS)
    partial = pl.kernel(
        _sc_gather_scatter,
        out_type=jax.ShapeDtypeStruct((NC, NPAD, D_OUT), jnp.float32),
        mesh=mesh,
        scratch_types=[
            pltpu.VMEM((CH,), jnp.int32),
            pltpu.VMEM((CH,), jnp.int32),
            pltpu.VMEM((CH, D_OUT), jnp.float32),
            pltpu.VMEM_SHARED((NPAD, D_OUT), jnp.float32),
            pltpu.SemaphoreType.DMA,
        ],
    )(hflat, gidx, dst, zinit)

    bias8 = jnp.broadcast_to(h_bias, (8, D_OUT))
    out = pl.pallas_call(
        _combine_body,
        grid=(N // NB,),
        in_specs=[
            pl.BlockSpec((NC, NB, D_OUT), lambda i: (0, i, 0)),
            pl.BlockSpec((NB, D_IN), lambda i: (i, 0)),
            pl.BlockSpec((D_IN, D_OUT), lambda i: (0, 0)),
            pl.BlockSpec((8, D_OUT), lambda i: (0, 0)),
        ],
        out_specs=pl.BlockSpec((NB, D_OUT), lambda i: (i, 0)),
        out_shape=jax.ShapeDtypeStruct((N, D_OUT), jnp.float32),
    )(partial, x, loop_weight, bias8)
    return out
